# Initial kernel scaffold; baseline (speedup 1.0000x reference)
import jax
import jax.numpy as jnp
from jax.experimental import pallas as pl


def kernel(x, adj, W1, W_mu, W_sigma, epsilon):
    raise NotImplementedError("write your pallas kernel here")


# speedup vs baseline: 11.9521x; 11.9521x over previous
import functools

import jax
import jax.numpy as jnp
from jax import lax
from jax.experimental import pallas as pl
from jax.experimental.pallas import tpu as pltpu
from jax.experimental.pallas import tpu_sc as plsc

N = 10000
E = 320000
D_IN = 128
H1 = 64
H2 = 32
W = 128

NC = 2
NS = 16
NW = NC * NS
CHUNK = 128
NCHUNK = -(-E // (NW * CHUNK))
E_PAD = NW * CHUNK * NCHUNK
N_PAD = 10240
RPS = N_PAD // NS
L = 16

_MESH = plsc.VectorSubcoreMesh(core_axis_name="c", subcore_axis_name="s")
_SC_PARAMS = pltpu.CompilerParams(needs_layout_passes=False)


def _deg_body(dst_hbm, out_hbm, dst_v, deg_v):
    c = lax.axis_index("c")
    s = lax.axis_index("s")
    wid = c * NS + s
    pltpu.sync_copy(dst_hbm.at[wid], dst_v)

    def zero(i, carry):
        deg_v[pl.ds(i * L, L)] = jnp.zeros((L,), jnp.float32)
        return carry

    lax.fori_loop(0, N_PAD // L, zero, 0)

    ones = jnp.ones((L,), jnp.float32)

    def body(i, carry):
        j = i // (CHUNK // L)
        k = i % (CHUNK // L)
        idx = dst_v[j, pl.ds(k * L, L)]
        plsc.addupdate_scatter(deg_v, [idx], ones)
        return carry

    lax.fori_loop(0, NCHUNK * (CHUNK // L), body, 0)
    pltpu.sync_copy(deg_v, out_hbm.at[wid])


@functools.partial(
    pl.kernel,
    mesh=_MESH,
    compiler_params=_SC_PARAMS,
    out_type=jax.ShapeDtypeStruct((NW, N_PAD), jnp.float32),
    scratch_types=[
        pltpu.VMEM((NCHUNK, CHUNK), jnp.int32),
        pltpu.VMEM((N_PAD,), jnp.float32),
    ],
)
def _deg_kernel(dst_hbm, out_hbm, dst_v, deg_v):
    _deg_body(dst_hbm, out_hbm, dst_v, deg_v)


def _spmm_body(h_hbm, src_hbm, dst_hbm, zeros_hbm, out_hbm,
               src_v, dst_v, rows_v, acc, sem):
    c = lax.axis_index("c")
    s = lax.axis_index("s")
    wid = c * NS + s
    pltpu.sync_copy(src_hbm.at[wid], src_v)
    pltpu.sync_copy(dst_hbm.at[wid], dst_v)
    pltpu.sync_copy(zeros_hbm, acc.at[pl.ds(s * RPS, RPS)])
    plsc.subcore_barrier()

    def body(j, carry):
        pltpu.async_copy(h_hbm.at[src_v.at[j]], rows_v, sem).wait()
        pltpu.sync_copy(rows_v, acc.at[dst_v.at[j]], add=True)
        return carry

    lax.fori_loop(0, NCHUNK, body, 0)
    plsc.subcore_barrier()
    pltpu.sync_copy(acc.at[pl.ds(s * RPS, RPS)],
                    out_hbm.at[c, pl.ds(s * RPS, RPS)])


@functools.partial(
    pl.kernel,
    mesh=_MESH,
    compiler_params=_SC_PARAMS,
    out_type=jax.ShapeDtypeStruct((NC, N_PAD, W), jnp.float32),
    scratch_types=[
        pltpu.VMEM((NCHUNK, CHUNK), jnp.int32),
        pltpu.VMEM((NCHUNK, CHUNK), jnp.int32),
        pltpu.VMEM((CHUNK, W), jnp.float32),
        pltpu.VMEM_SHARED((N_PAD, W), jnp.float32),
        pltpu.SemaphoreType.DMA,
    ],
)
def _spmm_kernel(h_hbm, src_hbm, dst_hbm, zeros_hbm, out_hbm,
                 src_v, dst_v, rows_v, acc, sem):
    _spmm_body(h_hbm, src_hbm, dst_hbm, zeros_hbm, out_hbm,
               src_v, dst_v, rows_v, acc, sem)


def _tc1_body(x_ref, w1_ref, deg_ref, h_ref, inv_ref):
    dsum = jnp.sum(deg_ref[...], axis=1, keepdims=True)
    inv = lax.rsqrt(jnp.maximum(dsum, 1.0))
    inv_ref[...] = inv
    h_ref[...] = jnp.dot(x_ref[...], w1_ref[...],
                         preferred_element_type=jnp.float32) * inv


def _tc2_body(p0_ref, p1_ref, inv_ref, w_ref, out_ref):
    inv = inv_ref[...]
    hidden = jnp.maximum((p0_ref[...] + p1_ref[...]) * inv, 0.0)
    out_ref[...] = jnp.dot(hidden, w_ref[...],
                           preferred_element_type=jnp.float32) * inv


def _tc3_body(zm0_ref, zm1_ref, zl0_ref, zl1_ref, inv_ref, eps_ref, z_ref):
    inv = inv_ref[...]
    zm = (zm0_ref[...] + zm1_ref[...]) * inv
    zl = (zl0_ref[...] + zl1_ref[...]) * inv
    z_ref[...] = zm + jnp.exp(0.5 * zl) * eps_ref[...]


def kernel(x, adj, W1, W_mu, W_sigma, epsilon):
    src = adj[0]
    dst = adj[1]
    pad = E_PAD - E
    src3 = jnp.concatenate([src, jnp.zeros((pad,), jnp.int32)]).reshape(
        NW, NCHUNK, CHUNK)
    dst3 = jnp.concatenate([dst, jnp.full((pad,), N, jnp.int32)]).reshape(
        NW, NCHUNK, CHUNK)

    zeros_h = jnp.zeros((RPS, W), jnp.float32)
    W1p = jnp.pad(W1, ((0, 0), (0, W - H1)))
    Wcat = jnp.pad(jnp.concatenate([W_mu, W_sigma], axis=1),
                   ((0, W - H1), (0, W - H1)))

    deg_parts = _deg_kernel(dst3)
    degT = deg_parts.T[:N]

    h0s, inv = pl.pallas_call(
        _tc1_body,
        out_shape=[
            jax.ShapeDtypeStruct((N, W), jnp.float32),
            jax.ShapeDtypeStruct((N, 1), jnp.float32),
        ],
    )(x, W1p, degT)

    p = _spmm_kernel(h0s, src3, dst3, zeros_h)

    h2s = pl.pallas_call(
        _tc2_body,
        out_shape=jax.ShapeDtypeStruct((N, W), jnp.float32),
    )(p[0, :N], p[1, :N], inv, Wcat)

    q = _spmm_kernel(h2s, src3, dst3, zeros_h)

    Z = pl.pallas_call(
        _tc3_body,
        out_shape=jax.ShapeDtypeStruct((N, H2), jnp.float32),
    )(q[0, :N, :H2], q[1, :N, :H2], q[0, :N, H2:H1], q[1, :N, H2:H1],
      inv, epsilon)
    return Z
